# Initial kernel scaffold; baseline (speedup 1.0000x reference)
import jax
import jax.numpy as jnp
from jax.experimental import pallas as pl


def kernel(image, rois):
    raise NotImplementedError("write your pallas kernel here")


# speedup vs baseline: 8.1736x; 8.1736x over previous
import jax
import jax.numpy as jnp
from jax.experimental import pallas as pl
from jax.experimental.pallas import tpu as pltpu

_P = 7
_STRIDE = 16.0
_B = 8
_HW = 50
_C = 512
_ROWS = 2560


def _axis(start, size, limit):
    i = jnp.arange(_P, dtype=jnp.float32)
    loc = (i[None, :] + 0.5) * (size[:, None] / _P) - 0.5
    loc = jnp.clip(loc, 0.0, size[:, None] - 1.0)
    lo = jnp.floor(loc)
    frac = loc - lo
    i0 = lo.astype(jnp.int32) + start[:, None].astype(jnp.int32)
    i0 = jnp.clip(i0, 0, limit - 1)
    return i0, frac


def _roi_body(idx_ref, w_ref, img_ref, out_ref):
    def per_roi(n, carry):
        wrow = w_ref[n]
        fx = [wrow[0:1, qq:qq + 1] for qq in range(_P)]
        fy = [wrow[0:1, _P + pp:_P + pp + 1] for pp in range(_P)]
        for p in range(_P):
            fyp = fy[p]
            for q in range(_P):
                cell = p * _P + q
                i0 = idx_ref[n, cell]
                g00 = img_ref[i0]
                g01 = img_ref[i0 + 1]
                g10 = img_ref[i0 + _HW]
                g11 = img_ref[i0 + _HW + 1]
                fxq = fx[q]
                top = g00 + fxq * (g01 - g00)
                bot = g10 + fxq * (g11 - g10)
                val = top + fyp * (bot - top)
                out_ref[n, :, _C * cell:_C * (cell + 1)] = val
        return carry

    jax.lax.fori_loop(0, _B, per_roi, 0)


def kernel(image, rois):
    n_rois = rois.shape[0]
    img = image[0].reshape(_HW * _HW, _C)
    img = jnp.pad(img, ((0, _ROWS - _HW * _HW), (0, 0)))
    img3 = img.reshape(_ROWS, 1, _C)

    q = jnp.round(rois / _STRIDE)
    y0, fy = _axis(q[:, 1], q[:, 3], _HW)
    x0, fx = _axis(q[:, 0], q[:, 2], _HW)
    idx = y0[:, :, None] * _HW + x0[:, None, :]
    idx = jnp.clip(idx, 0, _ROWS - _HW - 2).reshape(n_rois, _P * _P)
    idx = idx.astype(jnp.int32)
    wts = jnp.concatenate(
        [fx, fy, jnp.zeros((n_rois, 2), jnp.float32)], axis=1)
    wts = wts.astype(jnp.float32).reshape(n_rois, 1, 16)

    out = pl.pallas_call(
        _roi_body,
        grid=(n_rois // _B,),
        in_specs=[
            pl.BlockSpec((_B, _P * _P), lambda i: (i, 0),
                         memory_space=pltpu.SMEM),
            pl.BlockSpec((_B, 1, 16), lambda i: (i, 0, 0)),
            pl.BlockSpec((_ROWS, 1, _C), lambda i: (0, 0, 0)),
        ],
        out_specs=pl.BlockSpec((_B, 1, _P * _P * _C), lambda i: (i, 0, 0)),
        out_shape=jax.ShapeDtypeStruct((n_rois, 1, _P * _P * _C),
                                       jnp.float32),
        compiler_params=pltpu.CompilerParams(
            dimension_semantics=("parallel",),
        ),
    )(idx, wts, img3)
    return out.reshape(1, n_rois, _P, _P, _C)
